# Initial kernel scaffold; baseline (speedup 1.0000x reference)
import jax
import jax.numpy as jnp
from jax.experimental import pallas as pl


def kernel(x, edge_index, batch, W1, b1, W2, b2, W3, b3, W4, b4, W5, b5, fc1_W, fc1_b, fc2_W, fc2_b):
    raise NotImplementedError("write your pallas kernel here")


# speedup vs baseline: 1.8924x; 1.8924x over previous
import functools

import jax
import jax.numpy as jnp
from jax import lax
from jax.experimental import pallas as pl
from jax.experimental.pallas import tpu as pltpu
from jax.experimental.pallas import tpu_sc as plsc

N = 10000
E = 320000
D = 128
H = 64
G = 64
HW = 128

NC = 2
NS = 16
NW = NC * NS
CHUNK = 128
CPT = 80
EPT = CPT * CHUNK
EPAD = NW * EPT

HALF = 5120
NLO = HALF
NHI = N - HALF
NPAD = 6144
RPT = NPAD // NS
SINK = 6000
PAD_DST = HALF + SINK

_mesh = functools.partial(
    plsc.VectorSubcoreMesh, core_axis_name="c", subcore_axis_name="s"
)


def _sc_scatter(table, idx2d, row2d):
    @functools.partial(
        pl.kernel,
        out_type=jax.ShapeDtypeStruct((NC, NPAD, HW), jnp.float32),
        mesh=_mesh(),
        scratch_types=[
            pltpu.VMEM((CPT, CHUNK), jnp.int32),
            pltpu.VMEM((CPT, CHUNK), jnp.int32),
            pltpu.VMEM((CHUNK, HW), jnp.float32),
            pltpu.VMEM((CHUNK, HW), jnp.float32),
            pltpu.VMEM_SHARED((NPAD, HW), jnp.float32),
            pltpu.SemaphoreType.DMA,
            pltpu.SemaphoreType.DMA,
        ],
    )
    def k(tab_hbm, src_hbm, dst_hbm, out_hbm,
          srcv, dstv, gbuf0, gbuf1, acc, sem0, sem1):
        c = lax.axis_index("c")
        s = lax.axis_index("s")
        tile = c * NS + s

        @pl.loop(0, CHUNK)
        def _(i):
            @pl.loop(0, HW, step=16)
            def _(l):
                gbuf0[i, pl.ds(l, 16)] = jnp.zeros((16,), jnp.float32)

        @pl.loop(0, RPT, step=CHUNK)
        def _(r):
            pltpu.sync_copy(gbuf0, acc.at[pl.ds(s * RPT + r, CHUNK)])

        pltpu.sync_copy(src_hbm.at[pl.ds(tile * CPT, CPT)], srcv)
        pltpu.sync_copy(dst_hbm.at[pl.ds(tile * CPT, CPT)], dstv)
        plsc.subcore_barrier()

        pltpu.async_copy(tab_hbm.at[srcv.at[0]], gbuf0, sem0)
        pltpu.async_copy(tab_hbm.at[srcv.at[1]], gbuf1, sem1)

        @pl.loop(0, CPT, step=2)
        def _(j):
            pltpu.make_async_copy(tab_hbm.at[srcv.at[0]], gbuf0, sem0).wait()
            pltpu.sync_copy(gbuf0, acc.at[dstv.at[j]], add=True)

            @pl.when(j + 2 < CPT)
            def _():
                pltpu.async_copy(tab_hbm.at[srcv.at[j + 2]], gbuf0, sem0)

            pltpu.make_async_copy(tab_hbm.at[srcv.at[1]], gbuf1, sem1).wait()
            pltpu.sync_copy(gbuf1, acc.at[dstv.at[j + 1]], add=True)

            @pl.when(j + 3 < CPT)
            def _():
                pltpu.async_copy(tab_hbm.at[srcv.at[j + 3]], gbuf1, sem1)

        plsc.subcore_barrier()
        pltpu.sync_copy(acc.at[pl.ds(s * RPT, RPT)],
                        out_hbm.at[c, pl.ds(s * RPT, RPT)])

    return k(table, idx2d, row2d)


def _tc_prep(src2d, dst2d):
    def body(src_ref, dst_ref, idx_ref, row_ref, hi_ref):
        src = src_ref[...]
        dst = dst_ref[...]
        hi = (dst >= HALF).astype(jnp.int32)
        idx_ref[...] = src + N * hi
        row_ref[...] = dst - HALF * hi
        hi_ref[...] = hi

    shp = jax.ShapeDtypeStruct((NW * CPT, CHUNK), jnp.int32)
    return pl.pallas_call(body, out_shape=[shp, shp, shp])(src2d, dst2d)


def _unfold(s_ref, g_lo):
    ss = s_ref[0] + s_ref[1]
    return jnp.concatenate([ss[:NLO, :H], ss[:NHI, H:]], axis=0) + g_lo


def _tc_first(cnt, x, W1):
    def body(cnt_ref, x_ref, w_ref, dinv_ref, t_ref):
        cl = cnt_ref[0, :NLO, 0:1] + cnt_ref[1, :NLO, 0:1]
        ch = cnt_ref[0, :NHI, H:H + 1] + cnt_ref[1, :NHI, H:H + 1]
        deg = jnp.concatenate([cl, ch], axis=0) + 1.0
        dinv = lax.rsqrt(deg)
        g = jnp.dot(x_ref[...], w_ref[...], preferred_element_type=jnp.float32)
        g = g * dinv
        dinv_ref[...] = dinv
        z = jnp.zeros((N, H), jnp.float32)
        t_ref[:N, :H] = g
        t_ref[:N, H:] = z
        t_ref[N:, :H] = z
        t_ref[N:, H:] = g

    return pl.pallas_call(
        body,
        out_shape=[
            jax.ShapeDtypeStruct((N, 1), jnp.float32),
            jax.ShapeDtypeStruct((2 * N, HW), jnp.float32),
        ],
    )(cnt, x, W1)


def _tc_mid(sparts, t, dinv, b, Wn):
    def body(s_ref, t_ref, dinv_ref, b_ref, w_ref, out_ref):
        dinv = dinv_ref[...]
        h = _unfold(s_ref, t_ref[:N, :H])
        h = jnp.maximum(dinv * h + b_ref[...], 0.0)
        g = jnp.dot(h, w_ref[...], preferred_element_type=jnp.float32) * dinv
        z = jnp.zeros((N, H), jnp.float32)
        out_ref[:N, :H] = g
        out_ref[:N, H:] = z
        out_ref[N:, :H] = z
        out_ref[N:, H:] = g

    return pl.pallas_call(
        body,
        out_shape=jax.ShapeDtypeStruct((2 * N, HW), jnp.float32),
    )(sparts, t, dinv, b, Wn)


def _tc_final(sparts, t, dinv, b, batch2d, fc1_W, fc1_b, fc2_W, fc2_b):
    def body(s_ref, t_ref, dinv_ref, b_ref, batch_ref, f1w_ref, f1b_ref,
             f2w_ref, f2b_ref, out_ref):
        dinv = dinv_ref[...]
        h = _unfold(s_ref, t_ref[:N, :H])
        h = jnp.maximum(dinv * h + b_ref[...], 0.0)
        gids = lax.broadcasted_iota(jnp.int32, (1, G), 1)
        onehot = (batch_ref[...] == gids).astype(jnp.float32)
        pooled = lax.dot_general(
            onehot, h, (((0,), (0,)), ((), ())),
            preferred_element_type=jnp.float32,
        )
        p = jnp.maximum(
            jnp.dot(pooled, f1w_ref[...], preferred_element_type=jnp.float32)
            + f1b_ref[...],
            0.0,
        )
        out_ref[...] = (
            jnp.dot(p, f2w_ref[...], preferred_element_type=jnp.float32)
            + f2b_ref[...]
        )

    return pl.pallas_call(
        body,
        out_shape=jax.ShapeDtypeStruct((G, 1), jnp.float32),
    )(sparts, t, dinv, b, batch2d, fc1_W, fc1_b, fc2_W, fc2_b)


def kernel(x, edge_index, batch, W1, b1, W2, b2, W3, b3, W4, b4, W5, b5,
           fc1_W, fc1_b, fc2_W, fc2_b):
    pad = EPAD - E
    src2d = jnp.concatenate(
        [edge_index[0], jnp.zeros((pad,), jnp.int32)]).reshape(NW * CPT, CHUNK)
    dst2d = jnp.concatenate(
        [edge_index[1],
         jnp.full((pad,), PAD_DST, jnp.int32)]).reshape(NW * CPT, CHUNK)
    batch2d = batch.reshape(N, 1)
    ones_tab = jnp.concatenate(
        [jnp.ones((1, H), jnp.float32), jnp.zeros((1, H), jnp.float32)], axis=1)
    deg_tab = jnp.concatenate(
        [ones_tab, jnp.roll(ones_tab, H, axis=1),
         jnp.zeros((6, HW), jnp.float32)], axis=0)

    idx2d, row2d, hi2d = _tc_prep(src2d, dst2d)
    cnt = _sc_scatter(deg_tab, hi2d, row2d)
    dinv, t = _tc_first(cnt, x, W1)
    for Wn, b in ((W2, b1), (W3, b2), (W4, b3), (W5, b4)):
        s = _sc_scatter(t, idx2d, row2d)
        t = _tc_mid(s, t, dinv, b.reshape(1, H), Wn)
    s = _sc_scatter(t, idx2d, row2d)
    return _tc_final(s, t, dinv, b5.reshape(1, H), batch2d,
                     fc1_W, fc1_b.reshape(1, 32), fc2_W, fc2_b.reshape(1, 1))


# speedup vs baseline: 5.7866x; 3.0578x over previous
import functools

import jax
import jax.numpy as jnp
from jax import lax
from jax.experimental import pallas as pl
from jax.experimental.pallas import tpu as pltpu
from jax.experimental.pallas import tpu_sc as plsc

N = 10000
E = 320000
D = 128
H = 64
G = 64
HW = 128

NC = 2
NS = 16
NW = NC * NS
CHUNK = 128
CPT = 80
EPT = CPT * CHUNK
EPAD = NW * EPT

HALF = 5120
NLO = HALF
NHI = N - HALF
NPAD = 6144
RPT = NPAD // NS
SINK = 6000
PAD_DST = HALF + SINK
DEGR = 1024

_mesh = functools.partial(
    plsc.VectorSubcoreMesh, core_axis_name="c", subcore_axis_name="s"
)


def _sc_scatter(table, idx2d, row2d):
    @functools.partial(
        pl.kernel,
        out_type=jax.ShapeDtypeStruct((NC, NPAD, HW), jnp.float32),
        mesh=_mesh(),
        scratch_types=[
            pltpu.VMEM((CPT, CHUNK), jnp.int32),
            pltpu.VMEM((CPT, CHUNK), jnp.int32),
            pltpu.VMEM((CHUNK, HW), jnp.float32),
            pltpu.VMEM((CHUNK, HW), jnp.float32),
            pltpu.VMEM_SHARED((NPAD, HW), jnp.float32),
            pltpu.SemaphoreType.DMA,
            pltpu.SemaphoreType.DMA,
        ],
    )
    def k(tab_hbm, src_hbm, dst_hbm, out_hbm,
          srcv, dstv, gbuf0, gbuf1, acc, sem0, sem1):
        c = lax.axis_index("c")
        s = lax.axis_index("s")
        tile = c * NS + s

        @pl.loop(0, CHUNK)
        def _(i):
            @pl.loop(0, HW, step=16)
            def _(l):
                gbuf0[i, pl.ds(l, 16)] = jnp.zeros((16,), jnp.float32)

        @pl.loop(0, RPT, step=CHUNK)
        def _(r):
            pltpu.sync_copy(gbuf0, acc.at[pl.ds(s * RPT + r, CHUNK)])

        pltpu.sync_copy(src_hbm.at[pl.ds(tile * CPT, CPT)], srcv)
        pltpu.sync_copy(dst_hbm.at[pl.ds(tile * CPT, CPT)], dstv)
        plsc.subcore_barrier()

        pltpu.async_copy(tab_hbm.at[srcv.at[0]], gbuf0, sem0)
        pltpu.async_copy(tab_hbm.at[srcv.at[1]], gbuf1, sem1)

        @pl.loop(0, CPT, step=2)
        def _(j):
            pltpu.make_async_copy(tab_hbm.at[srcv.at[0]], gbuf0, sem0).wait()
            pltpu.sync_copy(gbuf0, acc.at[dstv.at[j]], add=True)

            @pl.when(j + 2 < CPT)
            def _():
                pltpu.async_copy(tab_hbm.at[srcv.at[j + 2]], gbuf0, sem0)

            pltpu.make_async_copy(tab_hbm.at[srcv.at[1]], gbuf1, sem1).wait()
            pltpu.sync_copy(gbuf1, acc.at[dstv.at[j + 1]], add=True)

            @pl.when(j + 3 < CPT)
            def _():
                pltpu.async_copy(tab_hbm.at[srcv.at[j + 3]], gbuf1, sem1)

        plsc.subcore_barrier()
        pltpu.sync_copy(acc.at[pl.ds(s * RPT, RPT)],
                        out_hbm.at[c, pl.ds(s * RPT, RPT)])

    return k(table, idx2d, row2d)


def _tc_prep(src2d, dst2d):
    def body(src_ref, dst_ref, idx_ref, row_ref, deg_ref):
        src = src_ref[...]
        dst = dst_ref[...]
        hi = (dst >= HALF).astype(jnp.int32)
        idx_ref[...] = src + N * hi
        row_ref[...] = dst - HALF * hi
        deg_ref[...] = hi * DEGR + (src & (DEGR - 1))

    shp = jax.ShapeDtypeStruct((NW * CPT, CHUNK), jnp.int32)
    return pl.pallas_call(body, out_shape=[shp, shp, shp])(src2d, dst2d)


def _unfold(s_ref, g_lo):
    ss = s_ref[0] + s_ref[1]
    return jnp.concatenate([ss[:NLO, :H], ss[:NHI, H:]], axis=0) + g_lo


def _tc_first(cnt, x, W1):
    def body(cnt_ref, x_ref, w_ref, dinv_ref, t_ref):
        cl = cnt_ref[0, :NLO, 0:1] + cnt_ref[1, :NLO, 0:1]
        ch = cnt_ref[0, :NHI, H:H + 1] + cnt_ref[1, :NHI, H:H + 1]
        deg = jnp.concatenate([cl, ch], axis=0) + 1.0
        dinv = lax.rsqrt(deg)
        g = jnp.dot(x_ref[...], w_ref[...], preferred_element_type=jnp.float32)
        g = g * dinv
        dinv_ref[...] = dinv
        z = jnp.zeros((N, H), jnp.float32)
        t_ref[:N, :H] = g
        t_ref[:N, H:] = z
        t_ref[N:, :H] = z
        t_ref[N:, H:] = g

    return pl.pallas_call(
        body,
        out_shape=[
            jax.ShapeDtypeStruct((N, 1), jnp.float32),
            jax.ShapeDtypeStruct((2 * N, HW), jnp.float32),
        ],
    )(cnt, x, W1)


def _tc_mid(sparts, t, dinv, b, Wn):
    def body(s_ref, t_ref, dinv_ref, b_ref, w_ref, out_ref):
        dinv = dinv_ref[...]
        h = _unfold(s_ref, t_ref[:N, :H])
        h = jnp.maximum(dinv * h + b_ref[...], 0.0)
        g = jnp.dot(h, w_ref[...], preferred_element_type=jnp.float32) * dinv
        z = jnp.zeros((N, H), jnp.float32)
        out_ref[:N, :H] = g
        out_ref[:N, H:] = z
        out_ref[N:, :H] = z
        out_ref[N:, H:] = g

    return pl.pallas_call(
        body,
        out_shape=jax.ShapeDtypeStruct((2 * N, HW), jnp.float32),
    )(sparts, t, dinv, b, Wn)


def _tc_final(sparts, t, dinv, b, batch2d, fc1_W, fc1_b, fc2_W, fc2_b):
    def body(s_ref, t_ref, dinv_ref, b_ref, batch_ref, f1w_ref, f1b_ref,
             f2w_ref, f2b_ref, out_ref):
        dinv = dinv_ref[...]
        h = _unfold(s_ref, t_ref[:N, :H])
        h = jnp.maximum(dinv * h + b_ref[...], 0.0)
        gids = lax.broadcasted_iota(jnp.int32, (1, G), 1)
        onehot = (batch_ref[...] == gids).astype(jnp.float32)
        pooled = lax.dot_general(
            onehot, h, (((0,), (0,)), ((), ())),
            preferred_element_type=jnp.float32,
        )
        p = jnp.maximum(
            jnp.dot(pooled, f1w_ref[...], preferred_element_type=jnp.float32)
            + f1b_ref[...],
            0.0,
        )
        out_ref[...] = (
            jnp.dot(p, f2w_ref[...], preferred_element_type=jnp.float32)
            + f2b_ref[...]
        )

    return pl.pallas_call(
        body,
        out_shape=jax.ShapeDtypeStruct((G, 1), jnp.float32),
    )(sparts, t, dinv, b, batch2d, fc1_W, fc1_b, fc2_W, fc2_b)


def kernel(x, edge_index, batch, W1, b1, W2, b2, W3, b3, W4, b4, W5, b5,
           fc1_W, fc1_b, fc2_W, fc2_b):
    pad = EPAD - E
    src2d = jnp.concatenate(
        [edge_index[0], jnp.zeros((pad,), jnp.int32)]).reshape(NW * CPT, CHUNK)
    dst2d = jnp.concatenate(
        [edge_index[1],
         jnp.full((pad,), PAD_DST, jnp.int32)]).reshape(NW * CPT, CHUNK)
    batch2d = batch.reshape(N, 1)
    ones_tab = jnp.concatenate(
        [jnp.ones((1, H), jnp.float32), jnp.zeros((1, H), jnp.float32)], axis=1)
    deg_tab = jnp.concatenate(
        [jnp.tile(ones_tab, (DEGR, 1)),
         jnp.tile(jnp.roll(ones_tab, H, axis=1), (DEGR, 1))], axis=0)

    idx2d, row2d, deg2d = _tc_prep(src2d, dst2d)
    cnt = _sc_scatter(deg_tab, deg2d, row2d)
    dinv, t = _tc_first(cnt, x, W1)
    for Wn, b in ((W2, b1), (W3, b2), (W4, b3), (W5, b4)):
        s = _sc_scatter(t, idx2d, row2d)
        t = _tc_mid(s, t, dinv, b.reshape(1, H), Wn)
    s = _sc_scatter(t, idx2d, row2d)
    return _tc_final(s, t, dinv, b5.reshape(1, H), batch2d,
                     fc1_W, fc1_b.reshape(1, 32), fc2_W, fc2_b.reshape(1, 1))
